# baseline (speedup 1.0000x reference)
import jax
import jax.numpy as jnp
from jax import lax
from jax.experimental import pallas as pl

B = 16384
D = 128
H = 256
R = 4
RD = 64
NV = 2
DK1 = 16
DK2 = 32

TB = 4096


def _bdot(a, b):
    return jnp.dot(a.astype(jnp.bfloat16), b.astype(jnp.bfloat16),
                   preferred_element_type=jnp.float32)


def _tables_kernel(re_ref, wproj_ref, bproj_ref, wk1_ref, wq2_ref,
                   rp_ref, k1_ref, q2t_ref):
    re = re_ref[:, :]
    rp = _bdot(re, wproj_ref[:, :]) + bproj_ref[:, :]
    rp_ref[:, :] = rp
    for r in range(R):
        k1_ref[r:r + 1, :] = _bdot(rp[r:r + 1, :], wk1_ref[r])
    q2t_ref[:, :] = _bdot(re, wq2_ref[:, :])


def _main_kernel(hid_ref, wenc_ref, benc_ref, wq1_ref, rp_ref, k1_ref,
                 q2t_ref, wvr1_ref, bvr1_ref, wvr2_ref, bvr2_ref,
                 out_ref):
    h0 = hid_ref[0]
    h1 = hid_ref[1]
    h2 = hid_ref[2]

    hcat = jnp.concatenate([h0, h2, h1, h2], axis=1)
    big = _bdot(hcat, wenc_ref[:, :])
    state = big[:, 0:H] + benc_ref[:, :]
    k2_0 = big[:, H:H + DK2]
    k2_1 = big[:, H + DK2:H + 2 * DK2]

    q1 = _bdot(state, wq1_ref[:, :])
    k1 = k1_ref[:, :]
    s1 = jnp.concatenate(
        [jnp.sum(q1 * k1[r:r + 1, :], axis=1, keepdims=True)
         for r in range(R)], axis=1) * 0.25
    m1 = jnp.max(s1, axis=1, keepdims=True)
    ii = lax.broadcasted_iota(jnp.int32, s1.shape, 1)
    idx = jnp.min(jnp.where(s1 == m1, ii, R), axis=1, keepdims=True)
    rmask = (ii == idx).astype(jnp.float32)

    sel_rp = _bdot(rmask, rp_ref[:, :])
    q2t = q2t_ref[:, :]
    q2 = jnp.where(idx == 0, q2t[0:1, :],
                   jnp.where(idx == 1, q2t[1:2, :],
                             jnp.where(idx == 2, q2t[2:3, :], q2t[3:4, :])))

    s2_0 = jnp.sum(q2 * k2_0, axis=1, keepdims=True)
    s2_1 = jnp.sum(q2 * k2_1, axis=1, keepdims=True)
    take1 = s2_1 > s2_0
    sel_lo = jnp.where(take1, h1, h0)

    wvr1 = wvr1_ref[:, :]
    pre = (_bdot(sel_lo, wvr1[0:D, :]) + _bdot(h2, wvr1[D:2 * D, :]) +
           _bdot(sel_rp, wvr1[2 * D:4 * D, :]) +
           _bdot(state, wvr1[4 * D:6 * D, :]) + bvr1_ref[:, :])
    hdn = jnp.maximum(pre, 0.0)
    out_ref[:, :] = _bdot(hdn, wvr2_ref[:, :]) + bvr2_ref[:, :]


def kernel(hidden, rule_emb, Wproj, bproj, Wenc, benc, Wq1, Wk1, Wq2, Wk2,
           Wvr1, bvr1, Wvr2, bvr2):
    Bsz = hidden.shape[0]
    hidden_t = jnp.transpose(hidden, (1, 0, 2))
    re2d = rule_emb.reshape(R, RD)
    z = jnp.zeros((D, DK2), jnp.float32)
    k2block = jnp.concatenate([
        jnp.concatenate([Wk2[0, 0:D], z], axis=1),
        jnp.concatenate([Wk2[0, D:2 * D], z], axis=1),
        jnp.concatenate([z, Wk2[1, 0:D]], axis=1),
        jnp.concatenate([z, Wk2[1, D:2 * D]], axis=1),
    ], axis=0)
    wenc_big = jnp.concatenate([Wenc, k2block], axis=1)
    bproj2 = bproj.reshape(1, H)
    benc2 = benc.reshape(1, H)
    bvr12 = bvr1.reshape(1, 128)
    bvr22 = bvr2.reshape(1, H)

    full = lambda shape: pl.BlockSpec(shape, lambda *a: (0,) * len(shape))

    rp, k1, q2t = pl.pallas_call(
        _tables_kernel,
        out_shape=(
            jax.ShapeDtypeStruct((R, H), jnp.float32),
            jax.ShapeDtypeStruct((R, DK1), jnp.float32),
            jax.ShapeDtypeStruct((R, DK2), jnp.float32),
        ),
    )(re2d, Wproj, bproj2, Wk1, Wq2)

    grid = (Bsz // TB,)
    out = pl.pallas_call(
        _main_kernel,
        grid=grid,
        in_specs=[
            pl.BlockSpec((3, TB, D), lambda i: (0, i, 0)),
            full((4 * D, H + 2 * DK2)),
            full((1, H)),
            full((H, DK1)),
            full((R, H)),
            full((R, DK1)),
            full((R, DK2)),
            full((3 * H, 128)),
            full((1, 128)),
            full((128, H)),
            full((1, H)),
        ],
        out_specs=pl.BlockSpec((TB, H), lambda i: (i, 0)),
        out_shape=jax.ShapeDtypeStruct((Bsz, H), jnp.float32),
    )(hidden_t, wenc_big, benc2, Wq1, rp, k1, q2t, Wvr1, bvr12, Wvr2, bvr22)
    return out


# speedup vs baseline: 1.3027x; 1.3027x over previous
import jax
import jax.numpy as jnp
from jax import lax
from jax.experimental import pallas as pl
from jax.experimental.pallas import tpu as pltpu

B = 16384
D = 128
H = 256
R = 4
RD = 64
NV = 2
DK1 = 16
DK2 = 32

TB = 4096


def _bdot(a, b):
    return jnp.dot(a.astype(jnp.bfloat16), b.astype(jnp.bfloat16),
                   preferred_element_type=jnp.float32)


def _tables_kernel(re_ref, wproj_ref, bproj_ref, wk1_ref, wq2_ref,
                   rp_ref, k1_ref, q2t_ref):
    re = re_ref[:, :]
    rp = _bdot(re, wproj_ref[:, :]) + bproj_ref[:, :]
    rp_ref[:, :] = rp
    for r in range(R):
        k1_ref[r:r + 1, :] = _bdot(rp[r:r + 1, :], wk1_ref[r])
    q2t_ref[:, :] = _bdot(re, wq2_ref[:, :])


def _main_kernel(hid_ref, wenc_ref, benc_ref, wq1_ref, rp_ref, k1_ref,
                 q2t_ref, wk2_ref, wvr1_ref, bvr1_ref, wvr2_ref, bvr2_ref,
                 out_ref):
    h0 = hid_ref[0]
    h1 = hid_ref[1]
    h2 = hid_ref[2]

    hcat = jnp.concatenate([h0, h2, h1, h2], axis=1)
    state = _bdot(hcat, wenc_ref[:, :]) + benc_ref[:, :]

    q1 = _bdot(state, wq1_ref[:, :])
    k1 = k1_ref[:, :]
    s1 = jnp.concatenate(
        [jnp.sum(q1 * k1[r:r + 1, :], axis=1, keepdims=True)
         for r in range(R)], axis=1) * 0.25
    m1 = jnp.max(s1, axis=1, keepdims=True)
    ii = lax.broadcasted_iota(jnp.int32, s1.shape, 1)
    idx = jnp.min(jnp.where(s1 == m1, ii, R), axis=1, keepdims=True)
    rmask = (ii == idx).astype(jnp.float32)

    sel_rp = _bdot(rmask, rp_ref[:, :])
    q2t = q2t_ref[:, :]
    q2 = jnp.where(idx == 0, q2t[0:1, :],
                   jnp.where(idx == 1, q2t[1:2, :],
                             jnp.where(idx == 2, q2t[2:3, :], q2t[3:4, :])))

    k2_0 = _bdot(h0, wk2_ref[0, 0:D, :]) + _bdot(h2, wk2_ref[0, D:2 * D, :])
    k2_1 = _bdot(h1, wk2_ref[1, 0:D, :]) + _bdot(h2, wk2_ref[1, D:2 * D, :])
    s2_0 = jnp.sum(q2 * k2_0, axis=1, keepdims=True)
    s2_1 = jnp.sum(q2 * k2_1, axis=1, keepdims=True)
    take1 = s2_1 > s2_0
    sel_lo = jnp.where(take1, h1, h0)

    wvr1 = wvr1_ref[:, :]
    pre = (_bdot(sel_lo, wvr1[0:D, :]) + _bdot(h2, wvr1[D:2 * D, :]) +
           _bdot(sel_rp, wvr1[2 * D:4 * D, :]) +
           _bdot(state, wvr1[4 * D:6 * D, :]) + bvr1_ref[:, :])
    hdn = jnp.maximum(pre, 0.0)
    out_ref[:, :] = _bdot(hdn, wvr2_ref[:, :]) + bvr2_ref[:, :]


def kernel(hidden, rule_emb, Wproj, bproj, Wenc, benc, Wq1, Wk1, Wq2, Wk2,
           Wvr1, bvr1, Wvr2, bvr2):
    Bsz = hidden.shape[0]
    hidden_t = jnp.transpose(hidden, (1, 0, 2))
    re2d = rule_emb.reshape(R, RD)
    bproj2 = bproj.reshape(1, H)
    benc2 = benc.reshape(1, H)
    bvr12 = bvr1.reshape(1, 128)
    bvr22 = bvr2.reshape(1, H)

    full = lambda shape: pl.BlockSpec(shape, lambda *a: (0,) * len(shape))

    rp, k1, q2t = pl.pallas_call(
        _tables_kernel,
        out_shape=(
            jax.ShapeDtypeStruct((R, H), jnp.float32),
            jax.ShapeDtypeStruct((R, DK1), jnp.float32),
            jax.ShapeDtypeStruct((R, DK2), jnp.float32),
        ),
    )(re2d, Wproj, bproj2, Wk1, Wq2)

    grid = (Bsz // TB,)
    out = pl.pallas_call(
        _main_kernel,
        grid=grid,
        compiler_params=pltpu.CompilerParams(
            dimension_semantics=("parallel",)),
        in_specs=[
            pl.BlockSpec((3, TB, D), lambda i: (0, i, 0)),
            full((4 * D, H)),
            full((1, H)),
            full((H, DK1)),
            full((R, H)),
            full((R, DK1)),
            full((R, DK2)),
            full((NV, H, DK2)),
            full((3 * H, 128)),
            full((1, 128)),
            full((128, H)),
            full((1, H)),
        ],
        out_specs=pl.BlockSpec((TB, H), lambda i: (i, 0)),
        out_shape=jax.ShapeDtypeStruct((Bsz, H), jnp.float32),
    )(hidden_t, Wenc, benc2, Wq1, rp, k1, q2t, Wk2, Wvr1, bvr12, Wvr2, bvr22)
    return out
